# baseline (speedup 1.0000x reference)
import functools

import jax
import jax.numpy as jnp
import numpy as np
from jax import lax
from jax.experimental import pallas as pl
from jax.experimental.pallas import tpu as pltpu
from jax.experimental.pallas import tpu_sc as plsc

ROWS = 64
N = 32768
LANES = 16
VBATCH = 4
VL = LANES * VBATCH
SEGV = N // VL
NBINS = 256
NPASS = 4
WORKERS = 32
ROWS_PER_WORKER = ROWS // WORKERS
INT_MIN = np.int32(-2147483648)
KEYPAD = N + N // SEGV


def _phi(idx):
  return idx + lax.shift_right_logical(idx, 9)


def _row_sort_body(key, ia, ib, hist):
  iota = lax.iota(jnp.int32, LANES)
  vl_addr = [jnp.int32(j * LANES) + iota for j in range(VBATCH)]
  q_base = [v * SEGV for v in vl_addr]
  qphi_base = [v * (SEGV + 1) for v in vl_addr]
  ones = jnp.full((LANES,), 1, jnp.int32)
  zeros = jnp.zeros((LANES,), jnp.int32)

  def zero_hist():
    @plsc.parallel_loop(0, NBINS * VL // LANES, unroll=8)
    def _zero(j):
      hist[pl.ds(j * LANES, LANES)] = zeros

  zero_hist()

  @plsc.parallel_loop(0, N // LANES, unroll=8)
  def _xform(t):
    pos = t * LANES + iota
    v = ib[pl.ds(t * LANES, LANES)]
    flip = lax.shift_right_arithmetic(v, 31) | INT_MIN
    plsc.store_scatter(key, [_phi(pos)], v ^ flip)

  for p in range(NPASS):
    shift = 8 * p
    src = (ib, ia)[p % 2]
    dst = (ib, ia)[(p + 1) % 2]
    last = p == NPASS - 1

    if p > 0:
      zero_hist()

    @plsc.parallel_loop(0, SEGV, unroll=2)
    def _count(t, shift=shift, src=src, p=p):
      for j in range(VBATCH):
        sl = pl.ds(t * VL + j * LANES, LANES)
        idxv = (q_base[j] + t) if p == 0 else src[sl]
        k = plsc.load_gather(key, [(qphi_base[j] + t) if p == 0 else _phi(idxv)])
        d = lax.shift_right_logical(k, shift) & 255
        addr = d * VL + vl_addr[j]
        plsc.addupdate_scatter(hist, [addr], ones)
        src[sl] = (addr << 17) | idxv

    @plsc.parallel_loop(0, NBINS * VL // LANES, unroll=4, carry=jnp.int32(0))
    def _scan(j, carry):
      sl = pl.ds(j * LANES, LANES)
      v = hist[sl]
      cs = plsc.cumsum(v)
      hist[sl] = cs - v + carry
      return carry + jnp.max(cs)

    def permute(t, _, src=src, dst=dst, last=last):
      packed = [src[pl.ds(t * VL + j * LANES, LANES)] for j in range(VBATCH)]
      addrs = [lax.shift_right_logical(v, 17) for v in packed]
      bases = [plsc.load_gather(hist, [a]) for a in addrs]
      for j in range(VBATCH):
        plsc.store_scatter(hist, [addrs[j]], bases[j] + ones)
      for j in range(VBATCH):
        base = bases[j]
        if last:
          wpos = base
        else:
          wpos = ((base & (SEGV - 1)) * VL) | lax.shift_right_logical(base, 9)
        plsc.store_scatter(dst, [wpos], packed[j] & 131071)
      return _

    lax.fori_loop(0, SEGV, permute, None, unroll=4)

  @plsc.parallel_loop(0, N // LANES, unroll=8)
  def _emit(t):
    sl = pl.ds(t * LANES, LANES)
    idxv = ib[sl]
    k = plsc.load_gather(key, [_phi(idxv)])
    flip = (~lax.shift_right_arithmetic(k, 31)) | INT_MIN
    ia[sl] = k ^ flip


@functools.cache
def _make_sort_kernel():
  mesh = plsc.VectorSubcoreMesh(core_axis_name="c", subcore_axis_name="s")

  @functools.partial(
      pl.kernel,
      out_type=(
          jax.ShapeDtypeStruct((ROWS, N), jnp.int32),
          jax.ShapeDtypeStruct((ROWS, N), jnp.int32),
      ),
      mesh=mesh,
      compiler_params=pltpu.CompilerParams(needs_layout_passes=False),
      scratch_types=[
          pltpu.VMEM((KEYPAD,), jnp.int32),
          pltpu.VMEM((N,), jnp.int32),
          pltpu.VMEM((N,), jnp.int32),
          pltpu.VMEM((NBINS * VL,), jnp.int32),
      ],
  )
  def sort_kernel(x_hbm, vals_hbm, idx_hbm, key, ia, ib, hist):
    wid = lax.axis_index("s") * 2 + lax.axis_index("c")

    def do_row(i, _):
      r = wid * ROWS_PER_WORKER + i
      pltpu.sync_copy(x_hbm.at[r], ib)
      _row_sort_body(key, ia, ib, hist)
      pltpu.sync_copy(ia, vals_hbm.at[r])
      pltpu.sync_copy(ib, idx_hbm.at[r])
      return _

    lax.fori_loop(0, ROWS_PER_WORKER, do_row, None)

  return sort_kernel


def kernel(x):
  bits = lax.bitcast_convert_type(x, jnp.int32)
  vals_bits, idx = _make_sort_kernel()(bits)
  vals = lax.bitcast_convert_type(vals_bits, jnp.float32)
  ok = jnp.array(True)
  return vals, idx, ok


# speedup vs baseline: 1.0012x; 1.0012x over previous
import functools

import jax
import jax.numpy as jnp
import numpy as np
from jax import lax
from jax.experimental import pallas as pl
from jax.experimental.pallas import tpu as pltpu
from jax.experimental.pallas import tpu_sc as plsc

ROWS = 64
N = 32768
LANES = 16
VBATCH = 4
VL = LANES * VBATCH
SEGV = N // VL
NBINS = 256
NPASS = 4
WORKERS = 32
ROWS_PER_WORKER = ROWS // WORKERS
INT_MIN = np.int32(-2147483648)
KEYPAD = N + N // SEGV


def _phi(idx):
  return idx + lax.shift_right_logical(idx, 9)


def _row_sort_body(key, ia, ib, hist):
  iota = lax.iota(jnp.int32, LANES)
  vl_addr = [jnp.int32(j * LANES) + iota for j in range(VBATCH)]
  q_base = [v * SEGV for v in vl_addr]
  qphi_base = [v * (SEGV + 1) for v in vl_addr]
  ones = jnp.full((LANES,), 1, jnp.int32)
  zeros = jnp.zeros((LANES,), jnp.int32)

  def zero_hist():
    @plsc.parallel_loop(0, NBINS * VL // LANES, unroll=8)
    def _zero(j):
      hist[pl.ds(j * LANES, LANES)] = zeros

  zero_hist()

  @plsc.parallel_loop(0, N // LANES, unroll=8)
  def _xform(t):
    pos = t * LANES + iota
    v = ib[pl.ds(t * LANES, LANES)]
    flip = lax.shift_right_arithmetic(v, 31) | INT_MIN
    plsc.store_scatter(key, [_phi(pos)], v ^ flip)

  for p in range(NPASS):
    shift = 8 * p
    src = (ib, ia)[p % 2]
    dst = (ib, ia)[(p + 1) % 2]
    last = p == NPASS - 1

    if p > 0:
      zero_hist()

    @plsc.parallel_loop(0, SEGV, unroll=2)
    def _count(t, shift=shift, src=src, p=p):
      for j in range(VBATCH):
        sl = pl.ds(t * VL + j * LANES, LANES)
        idxv = (q_base[j] + t) if p == 0 else src[sl]
        k = plsc.load_gather(key, [(qphi_base[j] + t) if p == 0 else _phi(idxv)])
        d = lax.shift_right_logical(k, shift) & 255
        addr = d * VL + vl_addr[j]
        plsc.addupdate_scatter(hist, [addr], ones)
        src[sl] = (addr << 17) | idxv

    @plsc.parallel_loop(0, NBINS * VL // LANES, unroll=4, carry=jnp.int32(0))
    def _scan(j, carry):
      sl = pl.ds(j * LANES, LANES)
      v = hist[sl]
      cs = plsc.cumsum(v)
      hist[sl] = cs - v + carry
      return carry + jnp.max(cs)

    def permute(t, _, src=src, dst=dst, last=last):
      packed = [src[pl.ds(t * VL + j * LANES, LANES)] for j in range(VBATCH)]
      addrs = [lax.shift_right_logical(v, 17) for v in packed]
      bases = [plsc.load_gather(hist, [a]) for a in addrs]
      for j in range(VBATCH):
        plsc.store_scatter(hist, [addrs[j]], bases[j] + ones)
      for j in range(VBATCH):
        base = bases[j]
        if last:
          wpos = base
        else:
          wpos = ((base & (SEGV - 1)) * VL) | lax.shift_right_logical(base, 9)
        plsc.store_scatter(dst, [wpos], packed[j] & 131071)
      return _

    lax.fori_loop(0, SEGV, permute, None, unroll=2)

  @plsc.parallel_loop(0, N // LANES, unroll=8)
  def _emit(t):
    sl = pl.ds(t * LANES, LANES)
    idxv = ib[sl]
    k = plsc.load_gather(key, [_phi(idxv)])
    flip = (~lax.shift_right_arithmetic(k, 31)) | INT_MIN
    ia[sl] = k ^ flip


@functools.cache
def _make_sort_kernel():
  mesh = plsc.VectorSubcoreMesh(core_axis_name="c", subcore_axis_name="s")

  @functools.partial(
      pl.kernel,
      out_type=(
          jax.ShapeDtypeStruct((ROWS, N), jnp.int32),
          jax.ShapeDtypeStruct((ROWS, N), jnp.int32),
      ),
      mesh=mesh,
      compiler_params=pltpu.CompilerParams(needs_layout_passes=False),
      scratch_types=[
          pltpu.VMEM((KEYPAD,), jnp.int32),
          pltpu.VMEM((N,), jnp.int32),
          pltpu.VMEM((N,), jnp.int32),
          pltpu.VMEM((NBINS * VL,), jnp.int32),
      ],
  )
  def sort_kernel(x_hbm, vals_hbm, idx_hbm, key, ia, ib, hist):
    wid = lax.axis_index("s") * 2 + lax.axis_index("c")

    def do_row(i, _):
      r = wid * ROWS_PER_WORKER + i
      pltpu.sync_copy(x_hbm.at[r], ib)
      _row_sort_body(key, ia, ib, hist)
      pltpu.sync_copy(ia, vals_hbm.at[r])
      pltpu.sync_copy(ib, idx_hbm.at[r])
      return _

    lax.fori_loop(0, ROWS_PER_WORKER, do_row, None)

  return sort_kernel


def kernel(x):
  bits = lax.bitcast_convert_type(x, jnp.int32)
  vals_bits, idx = _make_sort_kernel()(bits)
  vals = lax.bitcast_convert_type(vals_bits, jnp.float32)
  ok = jnp.array(True)
  return vals, idx, ok
